# baseline (speedup 1.0000x reference)
import functools

import jax
import jax.numpy as jnp
from jax import lax
from jax.experimental import pallas as pl
from jax.experimental.pallas import tpu as pltpu
from jax.experimental.pallas import tpu_sc as plsc

_SENT = 1.0e4
_INIT = 1.0e9


def _sc_sqrt(v):
    iv = plsc.bitcast(v, jnp.int32)
    r = plsc.bitcast(0x5F3759DF - (iv >> 1), jnp.float32)
    for _ in range(3):
        r = r * (1.5 - 0.5 * v * r * r)
    return v * r


def _sc_body(pred_hbm, lab_hbm, out_hbm, predv, labv, ffv, wfv, bnv, sbv,
             gtv, outv):
    io = lax.iota(jnp.int32, 16)
    wid = lax.axis_index("s") * 2 + lax.axis_index("c")
    m = wid % 16
    h = wid // 16
    p = m % 8
    dirn = m // 8
    jidx = p // 4
    i_img = p % 4

    pltpu.sync_copy(pred_hbm.at[pl.ds(i_img * 12288, 12288)], predv)
    pltpu.sync_copy(lab_hbm.at[pl.ds(i_img * 12288, 12288)], labv)

    jv = jnp.full((16,), jidx, jnp.int32)
    dv = jnp.full((16,), dirn, jnp.int32)

    def p1_body(y, carry):
        ca, cb, nw = carry
        for xg in range(4):
            off = y * 64 + xg * 16
            p0 = predv[pl.ds(off, 16)]
            p1 = predv[pl.ds(4096 + off, 16)]
            p2 = predv[pl.ds(8192 + off, 16)]
            b = labv[pl.ds((jidx + 1) * 4096 + off, 16)].astype(jnp.float32)
            oh1 = jnp.where((p1 > p0) & (p1 >= p2), 1.0, 0.0)
            oh2 = jnp.where((p2 > p0) & (p2 > p1), 1.0, 0.0)
            a = jnp.where(jv == 0, oh1, oh2)
            f_fld = jnp.where(dv == 0, b, a)
            g_fld = jnp.where(dv == 0, a, b)
            w = g_fld * (1.0 - f_fld)
            ffv[pl.ds(off, 16)] = f_fld
            wfv[pl.ds(off, 16)] = w
            ca = ca + a
            cb = cb + b
            nw = nw + w
        return ca, cb, nw

    z16 = jnp.zeros((16,), jnp.float32)
    ca_v, cb_v, nw_v = lax.fori_loop(0, 64, p1_body, (z16, z16, z16))

    def p2_body(y, ne):
        ym = jnp.maximum(y - 1, 0)
        yp = jnp.minimum(y + 1, 63)
        um = jnp.where(jnp.full((16,), y, jnp.int32) > 0, 1.0, 0.0)
        dm = jnp.where(jnp.full((16,), y, jnp.int32) < 63, 1.0, 0.0)
        for xg in range(4):
            gx = io + xg * 16
            lm = jnp.where(gx > 0, 1.0, 0.0)
            rm = jnp.where(gx < 63, 1.0, 0.0)
            off = y * 64 + xg * 16
            c = ffv[pl.ds(off, 16)]
            up = ffv[pl.ds(ym * 64 + xg * 16, 16)] * um
            dn = ffv[pl.ds(yp * 64 + xg * 16, 16)] * dm
            xi = off + io
            li = jnp.maximum(xi - 1, y * 64)
            ri = jnp.minimum(xi + 1, y * 64 + 63)
            lf = plsc.load_gather(ffv, [li]) * lm
            rf = plsc.load_gather(ffv, [ri]) * rm
            nb = c + up + dn + lf + rf
            bv = jnp.where(c * (5.0 - nb) > 0.0, 1.0, 0.0)
            bnv[pl.ds(off, 16)] = bv
            ne = ne + bv
        return ne

    ne_v = lax.fori_loop(0, 64, p2_body, z16)

    def p3_body(y, f):
        out = []
        for xg in range(4):
            off = y * 64 + xg * 16
            pen = bnv[pl.ds(off, 16)]
            fn = (f[xg] + 1.0) * (1.0 - pen)
            sbv[pl.ds(off, 16)] = fn
            out.append(fn)
        return tuple(out)

    s16 = jnp.full((16,), _SENT, jnp.float32)
    lax.fori_loop(0, 64, p3_body, (s16, s16, s16, s16))

    def p4_body(t, bw):
        y = 63 - t
        out = []
        for xg in range(4):
            off = y * 64 + xg * 16
            pen = bnv[pl.ds(off, 16)]
            bn = (bw[xg] + 1.0) * (1.0 - pen)
            near = jnp.minimum(bn, sbv[pl.ds(off, 16)])
            g = near * near
            idx = io * 64 + (xg * 1024 + y)
            plsc.store_scatter(gtv, [idx], g)
            out.append(bn)
        return tuple(out)

    lax.fori_loop(0, 64, p4_body, (s16, s16, s16, s16))

    x0 = h * 32

    def chunk_body(cidx, acc):
        xb = x0 + cidx * 8
        init = tuple(jnp.full((16,), _INIT, jnp.float32) for _ in range(32))

        def inner(xp, st):
            rows = [gtv[pl.ds(xp * 64 + q * 16, 16)] for q in range(4)]
            base = (xb - xp).astype(jnp.float32)
            new = []
            for k in range(8):
                dk = base + float(k)
                add = jnp.full((16,), dk * dk)
                for q in range(4):
                    new.append(jnp.minimum(st[k * 4 + q], rows[q] + add))
            return tuple(new)

        st = lax.fori_loop(0, 64, inner, init)
        for k in range(8):
            x = xb + k
            for q in range(4):
                s = _sc_sqrt(st[k * 4 + q])
                widx = io * 64 + (q * 1024 + x)
                wv = plsc.load_gather(wfv, [widx])
                acc = acc + s * wv
        return acc

    acc_v = lax.fori_loop(0, 4, chunk_body, z16)

    s_sum = jnp.sum(acc_v)
    res = jnp.where(io == 0, s_sum, 0.0)
    res = res + jnp.where(io == 1, jnp.sum(ne_v), 0.0)
    res = res + jnp.where(io == 2, jnp.sum(nw_v), 0.0)
    res = res + jnp.where(io == 3, jnp.sum(ca_v), 0.0)
    res = res + jnp.where(io == 4, jnp.sum(cb_v), 0.0)
    outv[...] = res
    pltpu.sync_copy(outv, out_hbm.at[pl.ds(wid * 16, 16)])


_sc_call = pl.kernel(
    _sc_body,
    out_type=jax.ShapeDtypeStruct((512,), jnp.float32),
    mesh=plsc.VectorSubcoreMesh(core_axis_name="c", subcore_axis_name="s",
                                num_cores=2, num_subcores=16),
    compiler_params=pltpu.CompilerParams(needs_layout_passes=False),
    scratch_types=[
        pltpu.VMEM((12288,), jnp.float32),
        pltpu.VMEM((12288,), jnp.int32),
        pltpu.VMEM((4096,), jnp.float32),
        pltpu.VMEM((4096,), jnp.float32),
        pltpu.VMEM((4096,), jnp.float32),
        pltpu.VMEM((4096,), jnp.float32),
        pltpu.VMEM((4096,), jnp.float32),
        pltpu.VMEM((16,), jnp.float32),
    ],
)


def _asm_body(p_ref, hd_ref, fail_ref):
    P = p_ref[...]
    hd = [None] * 8
    fail = [None] * 8
    for p in range(8):
        s_f = P[p, 0] + P[16 + p, 0]
        s_b = P[8 + p, 0] + P[24 + p, 0]
        ne_f = P[p, 1]
        nw_f = P[p, 2]
        ne_b = P[8 + p, 1]
        nw_b = P[8 + p, 2]
        ca = P[p, 3]
        cb = P[p, 4]
        hd_f = jnp.where((nw_f > 0) & (ne_f > 0),
                         s_f / jnp.maximum(ca, 1.0), 0.0)
        hd_b = jnp.where((nw_b > 0) & (ne_b > 0),
                         s_b / jnp.maximum(cb, 1.0), 0.0)
        hh = jnp.maximum(hd_f, hd_b)
        hd[p] = jnp.where(ca > 0, hh, 32.0)
        fail[p] = jnp.where(ca > 0, 0.0, 1.0)

    f1 = fail[0] + fail[1] + fail[2] + fail[3]
    f2 = fail[4] + fail[5] + fail[6] + fail[7]

    rr = lax.broadcasted_iota(jnp.int32, (8, 128), 0)
    cc = lax.broadcasted_iota(jnp.int32, (8, 128), 1)
    hdpad = jnp.zeros((8, 128), jnp.float32)
    for i in range(4):
        h1 = hd[i]
        h2 = hd[4 + i]
        for col, val in [(1, h1), (2, h2), (3, (h1 + h2) / 3.0),
                         (4, h1 / 2.0)]:
            hdpad = hdpad + jnp.where((rr == i) & (cc == col), val, 0.0)
    hd_ref[...] = hdpad

    fpad = jnp.zeros((8, 128), jnp.float32)
    for col, val in [(1, f1), (2, f2), (3, (f1 + f2) / 3.0),
                     (4, (f1 + f2) / 2.0)]:
        fpad = fpad + jnp.where((rr == 0) & (cc == col), val, 0.0)
    fail_ref[...] = fpad


def kernel(predictions, labels):
    partials = _sc_call(predictions.reshape(-1), labels.reshape(-1))
    hdpad, fpad = pl.pallas_call(
        _asm_body,
        out_shape=[
            jax.ShapeDtypeStruct((8, 128), jnp.float32),
            jax.ShapeDtypeStruct((8, 128), jnp.float32),
        ],
    )(partials.reshape(32, 16))
    return hdpad[:4, :5], fpad[0, :5]
